# baseline (speedup 1.0000x reference)
import functools

import jax
import jax.numpy as jnp
from jax import lax
from jax.experimental import pallas as pl
from jax.experimental.pallas import tpu as pltpu
from jax.experimental.pallas import tpu_sc as plsc

NUM_EMB = 1000000
DIM = 64
BATCH = 16384
SEQ = 50
B = BATCH * SEQ
NW = 32
BPW = B // NW
CHUNK = 800
NPAIR = BPW // (2 * CHUNK)

_mesh = plsc.VectorSubcoreMesh(core_axis_name="c", subcore_axis_name="s")


@functools.partial(
    pl.kernel,
    mesh=_mesh,
    out_type=jax.ShapeDtypeStruct((B, DIM), jnp.float32),
    scratch_types=[
        pltpu.VMEM((CHUNK,), jnp.int32),
        pltpu.VMEM((CHUNK,), jnp.int32),
        pltpu.VMEM((CHUNK, DIM), jnp.float32),
        pltpu.VMEM((CHUNK, DIM), jnp.float32),
        pltpu.SemaphoreType.DMA,
        pltpu.SemaphoreType.DMA,
    ],
    compiler_params=pltpu.CompilerParams(use_tc_tiling_on_sc=False),
)
def _embed_gather(table_hbm, idx_hbm, out_hbm,
                  idx_v0, idx_v1, rows_v0, rows_v1, sem0, sem1):
    wid = lax.axis_index("s") * 2 + lax.axis_index("c")
    base = wid * BPW

    def fetch(off, idx_v, rows_v, sem):
        pltpu.sync_copy(idx_hbm.at[pl.ds(off, CHUNK)], idx_v)
        pltpu.async_copy(table_hbm.at[idx_v], rows_v, sem)

    def drain(off, idx_v, rows_v, sem):
        pltpu.make_async_copy(table_hbm.at[idx_v], rows_v, sem).wait()
        pltpu.sync_copy(rows_v, out_hbm.at[pl.ds(off, CHUNK)])

    fetch(base, idx_v0, rows_v0, sem0)
    fetch(base + CHUNK, idx_v1, rows_v1, sem1)

    def body(i, carry):
        off0 = base + i * (2 * CHUNK)
        nxt0 = off0 + 2 * CHUNK
        drain(off0, idx_v0, rows_v0, sem0)

        @pl.when(i < NPAIR - 1)
        def _():
            fetch(nxt0, idx_v0, rows_v0, sem0)

        drain(off0 + CHUNK, idx_v1, rows_v1, sem1)

        @pl.when(i < NPAIR - 1)
        def _():
            fetch(nxt0 + CHUNK, idx_v1, rows_v1, sem1)

        return carry

    lax.fori_loop(0, NPAIR, body, 0)


def kernel(token_ids, embed_matrix):
    flat = token_ids.reshape(-1).astype(jnp.int32)
    out = _embed_gather(embed_matrix, flat)
    return out.reshape(token_ids.shape + (DIM,))
